# baseline (speedup 1.0000x reference)
import functools

import jax
import jax.numpy as jnp
from jax import lax
from jax.experimental import pallas as pl
from jax.experimental.pallas import tpu as pltpu
from jax.experimental.pallas import tpu_sc as plsc

VPAD = 1 << 20
NUM_IDS = 16384
DIM = 32
ROWS = 1024
LANES = 1024

_info = plsc.get_sparse_core_info()
_NC, _NS, _L = _info.num_cores, _info.num_subcores, _info.num_lanes
NW = _NC * _NS
CHUNK = VPAD // NW
IDS_PER_W = NUM_IDS // NW

_mesh = plsc.VectorSubcoreMesh(core_axis_name="c", subcore_axis_name="s")


NUM_ROWS = 1000000


@functools.partial(
    pl.kernel,
    mesh=_mesh,
    compiler_params=pltpu.CompilerParams(needs_layout_passes=False,
                                         use_tc_tiling_on_sc=False),
    out_type=jax.ShapeDtypeStruct((VPAD,), jnp.float32),
    scratch_types=[
        pltpu.VMEM((NUM_IDS,), jnp.int32),
        pltpu.VMEM((CHUNK,), jnp.float32),
    ],
)
def _sc_scatter(ids_hbm, p_hbm, allids_v, chunk_v):
    wid = lax.axis_index("s") * _NC + lax.axis_index("c")
    base = wid * CHUNK
    pltpu.sync_copy(ids_hbm, allids_v)
    zeros = jnp.zeros((_L,), jnp.float32)
    ones = jnp.ones((_L,), jnp.float32)

    def zbody(i, carry):
        chunk_v[pl.ds(i * _L, _L)] = zeros
        return carry

    lax.fori_loop(0, CHUNK // _L, zbody, 0, unroll=8)

    def sbody(i, carry):
        v = allids_v[pl.ds(i * _L, _L)]
        m = (v >= base) & (v < base + CHUNK)
        loc = jnp.where(m, v - base, 0)
        plsc.store_scatter(chunk_v, [loc], ones, mask=m)
        return carry

    lax.fori_loop(0, NUM_IDS // _L, sbody, 0, unroll=8)
    pltpu.sync_copy(chunk_v, p_hbm.at[pl.ds(base, CHUNK)])


def _tc_prefix_body(p_ref, d_ref):
    p = p_ref[...]
    ri = lax.broadcasted_iota(jnp.int32, (128, 128), 0)
    ci = lax.broadcasted_iota(jnp.int32, (128, 128), 1)
    m_incl = (ri <= ci).astype(jnp.bfloat16)
    c = jnp.dot(p.astype(jnp.bfloat16), m_incl,
                preferred_element_type=jnp.float32)
    r = c[:, 127:128]
    big_r = jnp.transpose(r).reshape(64, 128)
    c2 = jnp.dot(big_r.astype(jnp.bfloat16), m_incl,
                 preferred_element_type=jnp.float32)
    t = c2[:, 127:128]
    ri2 = lax.broadcasted_iota(jnp.int32, (64, 64), 0)
    ci2 = lax.broadcasted_iota(jnp.int32, (64, 64), 1)
    ms = (ri2 > ci2).astype(jnp.float32)
    e3 = jnp.dot(ms, t, preferred_element_type=jnp.float32,
                 precision=lax.Precision.HIGHEST)
    e2d = e3 + c2 - big_r
    e_col = jnp.transpose(e2d.reshape(1, 8192))
    d_ref[...] = (c + e_col).astype(jnp.int32) - 1


_tc_prefix = pl.pallas_call(
    _tc_prefix_body,
    out_shape=jax.ShapeDtypeStruct((VPAD // 128, 128), jnp.int32),
)


@functools.partial(
    pl.kernel,
    mesh=_mesh,
    compiler_params=pltpu.CompilerParams(needs_layout_passes=False),
    out_type=jax.ShapeDtypeStruct((NUM_IDS,), jnp.int32),
    scratch_types=[
        pltpu.VMEM((IDS_PER_W,), jnp.int32),
        pltpu.VMEM((IDS_PER_W,), jnp.int32),
        pltpu.SemaphoreType.DMA,
    ],
)
def _sc_rank_gather(ids_hbm, d_hbm, ret_hbm, idx_v, ret_v, sem):
    wid = lax.axis_index("s") * _NC + lax.axis_index("c")
    gbase = wid * IDS_PER_W
    pltpu.sync_copy(ids_hbm.at[pl.ds(gbase, IDS_PER_W)], idx_v)
    pltpu.async_copy(d_hbm.at[idx_v], ret_v, sem).wait()
    pltpu.sync_copy(ret_v, ret_hbm.at[pl.ds(gbase, IDS_PER_W)])


@functools.partial(
    pl.kernel,
    mesh=_mesh,
    compiler_params=pltpu.CompilerParams(needs_layout_passes=False,
                                         use_tc_tiling_on_sc=False),
    out_type=jax.ShapeDtypeStruct((NUM_IDS, DIM), jnp.float32),
    scratch_types=[
        pltpu.VMEM((IDS_PER_W,), jnp.int32),
        pltpu.VMEM((IDS_PER_W, DIM), jnp.float32),
        pltpu.SemaphoreType.DMA,
    ],
)
def _sc_wgather(ids_hbm, w_hbm, rows_hbm, idx_v, rows_v, sem):
    wid = lax.axis_index("s") * _NC + lax.axis_index("c")
    gbase = wid * IDS_PER_W
    pltpu.sync_copy(ids_hbm.at[pl.ds(gbase, IDS_PER_W)], idx_v)
    pltpu.async_copy(w_hbm.at[idx_v], rows_v, sem).wait()
    pltpu.sync_copy(rows_v, rows_hbm.at[pl.ds(gbase, IDS_PER_W)])


_TBLK = 8192


def _tc_relayout_body(x_ref, o_ref):
    x = x_ref[...]
    eye = (lax.broadcasted_iota(jnp.int32, (DIM, DIM), 0)
           == lax.broadcasted_iota(jnp.int32, (DIM, DIM), 1)).astype(jnp.float32)
    y = lax.dot_general(x, eye, (((0,), (0,)), ((), ())),
                        preferred_element_type=jnp.float32,
                        precision=lax.Precision.HIGHEST)
    y3 = y.reshape(_TBLK // 4, 4, DIM)
    o_ref[...] = jnp.concatenate(
        [y3[:, a, :] for a in range(4)], axis=1)


_tc_relayout = pl.pallas_call(
    _tc_relayout_body,
    grid=((NUM_ROWS + _TBLK - 1) // _TBLK,),
    in_specs=[pl.BlockSpec((DIM, _TBLK), lambda j: (0, j))],
    out_specs=pl.BlockSpec((_TBLK // 4, DIM * 4), lambda j: (j, 0)),
    out_shape=jax.ShapeDtypeStruct((NUM_ROWS // 4, DIM * 4), jnp.float32),
)


def kernel(ids, cpu_weight):
    w_lin = _tc_relayout(cpu_weight.T).reshape(NUM_ROWS, DIM)
    p = _sc_scatter(ids)
    out = _sc_wgather(ids, w_lin)
    d = _tc_prefix(p.reshape(VPAD // 128, 128))
    ret = _sc_rank_gather(ids, d.reshape(VPAD))
    return ret, out


# speedup vs baseline: 1.6598x; 1.6598x over previous
import functools

import jax
import jax.numpy as jnp
from jax import lax
from jax.experimental import pallas as pl
from jax.experimental.pallas import tpu as pltpu
from jax.experimental.pallas import tpu_sc as plsc

VPAD = 1 << 20
NUM_IDS = 16384
DIM = 32
ROWS = 1024
LANES = 1024

_info = plsc.get_sparse_core_info()
_NC, _NS, _L = _info.num_cores, _info.num_subcores, _info.num_lanes
NW = _NC * _NS
CHUNK = VPAD // NW
IDS_PER_W = NUM_IDS // NW

_mesh = plsc.VectorSubcoreMesh(core_axis_name="c", subcore_axis_name="s")


NUM_ROWS = 1000000


@functools.partial(
    pl.kernel,
    mesh=_mesh,
    compiler_params=pltpu.CompilerParams(needs_layout_passes=False,
                                         use_tc_tiling_on_sc=False),
    out_type=jax.ShapeDtypeStruct((VPAD,), jnp.float32),
    scratch_types=[
        pltpu.VMEM((NUM_IDS,), jnp.int32),
        pltpu.VMEM((CHUNK,), jnp.float32),
    ],
)
def _sc_scatter(ids_hbm, p_hbm, allids_v, chunk_v):
    wid = lax.axis_index("s") * _NC + lax.axis_index("c")
    base = wid * CHUNK
    pltpu.sync_copy(ids_hbm, allids_v)
    zeros = jnp.zeros((_L,), jnp.float32)
    ones = jnp.ones((_L,), jnp.float32)

    def zbody(i, carry):
        chunk_v[pl.ds(i * _L, _L)] = zeros
        return carry

    lax.fori_loop(0, CHUNK // _L, zbody, 0, unroll=8)

    def sbody(i, carry):
        v = allids_v[pl.ds(i * _L, _L)]
        m = (v >= base) & (v < base + CHUNK)
        loc = jnp.where(m, v - base, 0)
        plsc.store_scatter(chunk_v, [loc], ones, mask=m)
        return carry

    lax.fori_loop(0, NUM_IDS // _L, sbody, 0, unroll=8)
    pltpu.sync_copy(chunk_v, p_hbm.at[pl.ds(base, CHUNK)])


def _tc_prefix_body(p_ref, d_ref):
    p = p_ref[...]
    ri = lax.broadcasted_iota(jnp.int32, (128, 128), 0)
    ci = lax.broadcasted_iota(jnp.int32, (128, 128), 1)
    m_incl = (ri <= ci).astype(jnp.bfloat16)
    c = jnp.dot(p.astype(jnp.bfloat16), m_incl,
                preferred_element_type=jnp.float32)
    r = c[:, 127:128]
    big_r = jnp.transpose(r).reshape(64, 128)
    c2 = jnp.dot(big_r.astype(jnp.bfloat16), m_incl,
                 preferred_element_type=jnp.float32)
    t = c2[:, 127:128]
    ri2 = lax.broadcasted_iota(jnp.int32, (64, 64), 0)
    ci2 = lax.broadcasted_iota(jnp.int32, (64, 64), 1)
    ms = (ri2 > ci2).astype(jnp.float32)
    e3 = jnp.dot(ms, t, preferred_element_type=jnp.float32,
                 precision=lax.Precision.HIGHEST)
    e2d = e3 + c2 - big_r
    e_col = jnp.transpose(e2d.reshape(1, 8192))
    d_ref[...] = (c + e_col).astype(jnp.int32) - 1


_tc_prefix = pl.pallas_call(
    _tc_prefix_body,
    out_shape=jax.ShapeDtypeStruct((VPAD // 128, 128), jnp.int32),
)


@functools.partial(
    pl.kernel,
    mesh=_mesh,
    compiler_params=pltpu.CompilerParams(needs_layout_passes=False),
    out_type=jax.ShapeDtypeStruct((NUM_IDS,), jnp.int32),
    scratch_types=[
        pltpu.VMEM((IDS_PER_W,), jnp.int32),
        pltpu.VMEM((IDS_PER_W,), jnp.int32),
        pltpu.SemaphoreType.DMA,
    ],
)
def _sc_rank_gather(ids_hbm, d_hbm, ret_hbm, idx_v, ret_v, sem):
    wid = lax.axis_index("s") * _NC + lax.axis_index("c")
    gbase = wid * IDS_PER_W
    pltpu.sync_copy(ids_hbm.at[pl.ds(gbase, IDS_PER_W)], idx_v)
    pltpu.async_copy(d_hbm.at[idx_v], ret_v, sem).wait()
    pltpu.sync_copy(ret_v, ret_hbm.at[pl.ds(gbase, IDS_PER_W)])


@functools.partial(
    pl.kernel,
    mesh=_mesh,
    compiler_params=pltpu.CompilerParams(needs_layout_passes=False,
                                         use_tc_tiling_on_sc=False),
    out_type=jax.ShapeDtypeStruct((NUM_IDS, DIM), jnp.float32),
    scratch_types=[
        pltpu.VMEM((IDS_PER_W,), jnp.int32),
        pltpu.VMEM((IDS_PER_W, DIM), jnp.float32),
        pltpu.SemaphoreType.DMA,
    ],
)
def _sc_wgather(ids_hbm, w_hbm, rows_hbm, idx_v, rows_v, sem):
    wid = lax.axis_index("s") * _NC + lax.axis_index("c")
    gbase = wid * IDS_PER_W
    pltpu.sync_copy(ids_hbm.at[pl.ds(gbase, IDS_PER_W)], idx_v)
    pltpu.async_copy(w_hbm.at[idx_v], rows_v, sem).wait()
    pltpu.sync_copy(rows_v, rows_hbm.at[pl.ds(gbase, IDS_PER_W)])


_TBLK = 8192


def _tc_relayout_body(x_ref, o_ref):
    x = x_ref[...]
    y = jnp.transpose(x)
    y3 = y.reshape(_TBLK // 4, 4, DIM)
    o_ref[...] = jnp.concatenate(
        [y3[:, a, :] for a in range(4)], axis=1)


_tc_relayout = pl.pallas_call(
    _tc_relayout_body,
    grid=((NUM_ROWS + _TBLK - 1) // _TBLK,),
    in_specs=[pl.BlockSpec((DIM, _TBLK), lambda j: (0, j))],
    out_specs=pl.BlockSpec((_TBLK // 4, DIM * 4), lambda j: (j, 0)),
    out_shape=jax.ShapeDtypeStruct((NUM_ROWS // 4, DIM * 4), jnp.float32),
)


def kernel(ids, cpu_weight):
    w_lin = _tc_relayout(cpu_weight.T).reshape(NUM_ROWS, DIM)
    p = _sc_scatter(ids)
    out = _sc_wgather(ids, w_lin)
    d = _tc_prefix(p.reshape(VPAD // 128, 128))
    ret = _sc_rank_gather(ids, d.reshape(VPAD))
    return ret, out


# speedup vs baseline: 1.6855x; 1.0155x over previous
import functools

import jax
import jax.numpy as jnp
from jax import lax
from jax.experimental import pallas as pl
from jax.experimental.pallas import tpu as pltpu
from jax.experimental.pallas import tpu_sc as plsc

VPAD = 1 << 20
NUM_IDS = 16384
DIM = 32
ROWS = 1024
LANES = 1024

_info = plsc.get_sparse_core_info()
_NC, _NS, _L = _info.num_cores, _info.num_subcores, _info.num_lanes
NW = _NC * _NS
CHUNK = VPAD // NW
IDS_PER_W = NUM_IDS // NW

_mesh = plsc.VectorSubcoreMesh(core_axis_name="c", subcore_axis_name="s")


NUM_ROWS = 1000000


@functools.partial(
    pl.kernel,
    mesh=_mesh,
    compiler_params=pltpu.CompilerParams(needs_layout_passes=False,
                                         use_tc_tiling_on_sc=False),
    out_type=jax.ShapeDtypeStruct((VPAD,), jnp.float32),
    scratch_types=[
        pltpu.VMEM((NUM_IDS,), jnp.int32),
        pltpu.VMEM((CHUNK,), jnp.float32),
    ],
)
def _sc_scatter(ids_hbm, p_hbm, allids_v, chunk_v):
    wid = lax.axis_index("s") * _NC + lax.axis_index("c")
    base = wid * CHUNK
    pltpu.sync_copy(ids_hbm, allids_v)
    zeros = jnp.zeros((_L,), jnp.float32)
    ones = jnp.ones((_L,), jnp.float32)

    def zbody(i, carry):
        chunk_v[pl.ds(i * _L, _L)] = zeros
        return carry

    lax.fori_loop(0, CHUNK // _L, zbody, 0, unroll=8)

    def sbody(i, carry):
        v = allids_v[pl.ds(i * _L, _L)]
        m = (v >= base) & (v < base + CHUNK)
        loc = jnp.where(m, v - base, 0)
        plsc.store_scatter(chunk_v, [loc], ones, mask=m)
        return carry

    lax.fori_loop(0, NUM_IDS // _L, sbody, 0, unroll=8)
    pltpu.sync_copy(chunk_v, p_hbm.at[pl.ds(base, CHUNK)])


def _tc_prefix_body(p_ref, d_ref):
    p = p_ref[...]
    ri = lax.broadcasted_iota(jnp.int32, (128, 128), 0)
    ci = lax.broadcasted_iota(jnp.int32, (128, 128), 1)
    m_incl = (ri <= ci).astype(jnp.bfloat16)
    c = jnp.dot(p.astype(jnp.bfloat16), m_incl,
                preferred_element_type=jnp.float32)
    r = c[:, 127:128]
    big_r = jnp.transpose(r).reshape(64, 128)
    c2 = jnp.dot(big_r.astype(jnp.bfloat16), m_incl,
                 preferred_element_type=jnp.float32)
    t = c2[:, 127:128]
    ri2 = lax.broadcasted_iota(jnp.int32, (64, 64), 0)
    ci2 = lax.broadcasted_iota(jnp.int32, (64, 64), 1)
    ms = (ri2 > ci2).astype(jnp.float32)
    e3 = jnp.dot(ms, t, preferred_element_type=jnp.float32,
                 precision=lax.Precision.HIGHEST)
    e2d = e3 + c2 - big_r
    e_col = jnp.transpose(e2d.reshape(1, 8192))
    d_ref[...] = (c + e_col).astype(jnp.int32) - 1


_tc_prefix = pl.pallas_call(
    _tc_prefix_body,
    out_shape=jax.ShapeDtypeStruct((VPAD // 128, 128), jnp.int32),
)


@functools.partial(
    pl.kernel,
    mesh=_mesh,
    compiler_params=pltpu.CompilerParams(needs_layout_passes=False),
    out_type=jax.ShapeDtypeStruct((NUM_IDS,), jnp.int32),
    scratch_types=[
        pltpu.VMEM((IDS_PER_W,), jnp.int32),
        pltpu.VMEM((IDS_PER_W,), jnp.int32),
        pltpu.SemaphoreType.DMA,
    ],
)
def _sc_rank_gather(ids_hbm, d_hbm, ret_hbm, idx_v, ret_v, sem):
    wid = lax.axis_index("s") * _NC + lax.axis_index("c")
    gbase = wid * IDS_PER_W
    pltpu.sync_copy(ids_hbm.at[pl.ds(gbase, IDS_PER_W)], idx_v)
    pltpu.async_copy(d_hbm.at[idx_v], ret_v, sem).wait()
    pltpu.sync_copy(ret_v, ret_hbm.at[pl.ds(gbase, IDS_PER_W)])


@functools.partial(
    pl.kernel,
    mesh=_mesh,
    compiler_params=pltpu.CompilerParams(needs_layout_passes=False,
                                         use_tc_tiling_on_sc=False),
    out_type=jax.ShapeDtypeStruct((NUM_IDS, DIM), jnp.float32),
    scratch_types=[
        pltpu.VMEM((IDS_PER_W,), jnp.int32),
        pltpu.VMEM((IDS_PER_W, DIM), jnp.float32),
        pltpu.SemaphoreType.DMA,
    ],
)
def _sc_wgather(ids_hbm, w_hbm, rows_hbm, idx_v, rows_v, sem):
    wid = lax.axis_index("s") * _NC + lax.axis_index("c")
    gbase = wid * IDS_PER_W
    pltpu.sync_copy(ids_hbm.at[pl.ds(gbase, IDS_PER_W)], idx_v)
    pltpu.async_copy(w_hbm.at[idx_v], rows_v, sem).wait()
    pltpu.sync_copy(rows_v, rows_hbm.at[pl.ds(gbase, IDS_PER_W)])


_TBLK = 16384


def _tc_relayout_body(x_ref, o_ref):
    x = x_ref[...]
    y = jnp.transpose(x)
    y3 = y.reshape(_TBLK // 4, 4, DIM)
    o_ref[...] = jnp.concatenate(
        [y3[:, a, :] for a in range(4)], axis=1)


_tc_relayout = pl.pallas_call(
    _tc_relayout_body,
    grid=((NUM_ROWS + _TBLK - 1) // _TBLK,),
    in_specs=[pl.BlockSpec((DIM, _TBLK), lambda j: (0, j))],
    out_specs=pl.BlockSpec((_TBLK // 4, DIM * 4), lambda j: (j, 0)),
    out_shape=jax.ShapeDtypeStruct((NUM_ROWS // 4, DIM * 4), jnp.float32),
)


def kernel(ids, cpu_weight):
    w_lin = _tc_relayout(cpu_weight.T).reshape(NUM_ROWS, DIM)
    p = _sc_scatter(ids)
    out = _sc_wgather(ids, w_lin)
    d = _tc_prefix(p.reshape(VPAD // 128, 128))
    ret = _sc_rank_gather(ids, d.reshape(VPAD))
    return ret, out
